# Initial kernel scaffold; baseline (speedup 1.0000x reference)
import jax
import jax.numpy as jnp
from jax.experimental import pallas as pl


def kernel(latents, shape_emb, scale_emb, orient_emb, pos_x_emb, pos_y_emb, W1, b1, W2, b2):
    raise NotImplementedError("write your pallas kernel here")


# speedup vs baseline: 12.1012x; 12.1012x over previous
import functools

import jax
import jax.numpy as jnp
from jax.experimental import pallas as pl
from jax.experimental.pallas import tpu as pltpu

EMB = 128
BATCH = 16384
BLOCK = 2048
OFFS = (0, 8, 16, 56, 88)
ROWS = (3, 6, 40, 32, 32)


def _body(lat_ref, se_ref, sce_ref, oe_ref, xe_ref, ye_ref,
          w1_ref, b1_ref, w2_ref, b2_ref, out_ref, t_scratch):
    pid = pl.program_id(0)

    @pl.when(pid == 0)
    def _build_table():
        pieces = []
        for ref, off, rows, lo in zip(
                (se_ref, sce_ref, oe_ref, xe_ref, ye_ref), OFFS, ROWS,
                (0, 128, 256, 384, 512)):
            t = jax.lax.dot_general(
                ref[...], w1_ref[pl.ds(lo, EMB), :],
                (((1,), (0,)), ((), ())), preferred_element_type=jnp.float32)
            pad = (OFFS + (120,))[OFFS.index(off) + 1] - off - rows
            if pad:
                t = jnp.concatenate(
                    [t, jnp.zeros((pad, EMB), jnp.float32)], axis=0)
            pieces.append(t)
        pieces.append(jnp.zeros((8, EMB), jnp.float32))
        t_scratch[...] = jnp.concatenate(pieces, axis=0)

    lat = lat_ref[...]
    kio = jax.lax.broadcasted_iota(jnp.int32, (BLOCK, 128), 1)
    oh = jnp.zeros((BLOCK, 128), jnp.float32)
    for t, off in enumerate(OFFS):
        oh = oh + (kio == (lat[:, t + 1][:, None] + off)).astype(jnp.float32)
    g = jax.lax.dot_general(oh, t_scratch[...], (((1,), (0,)), ((), ())),
                            preferred_element_type=jnp.float32)
    g = g + b1_ref[...][None, :]
    h = g * jax.nn.sigmoid(g)
    o = jax.lax.dot_general(h, w2_ref[...], (((1,), (0,)), ((), ())),
                            preferred_element_type=jnp.float32)
    out_ref[...] = o + b2_ref[...][None, :]


@jax.jit
def kernel(latents, shape_emb, scale_emb, orient_emb, pos_x_emb, pos_y_emb,
           W1, b1, W2, b2):
    grid = BATCH // BLOCK
    full = lambda shape: pl.BlockSpec(shape, lambda i: (0,) * len(shape))
    return pl.pallas_call(
        _body,
        grid=(grid,),
        in_specs=[
            pl.BlockSpec((BLOCK, 6), lambda i: (i, 0)),
            full((3, EMB)), full((6, EMB)), full((40, EMB)),
            full((32, EMB)), full((32, EMB)),
            full((EMB * 5, EMB)), full((EMB,)),
            full((EMB, EMB)), full((EMB,)),
        ],
        out_specs=pl.BlockSpec((BLOCK, EMB), lambda i: (i, 0)),
        out_shape=jax.ShapeDtypeStruct((BATCH, EMB), jnp.float32),
        scratch_shapes=[pltpu.VMEM((128, EMB), jnp.float32)],
    )(latents, shape_emb, scale_emb, orient_emb, pos_x_emb, pos_y_emb,
      W1, b1, W2, b2)
